# baseline (speedup 1.0000x reference)
import functools

import jax
import jax.numpy as jnp
from jax import lax
from jax.experimental import pallas as pl
from jax.experimental.pallas import tpu as pltpu
from jax.experimental.pallas import tpu_sc as plsc

F32 = jnp.float32


def _zi():
    return jnp.int32(0)


def _mm_bias_body(x_ref, w_ref, b_ref, o_ref):
    o_ref[...] = (
        jnp.dot(x_ref[...], w_ref[...], preferred_element_type=F32,
                precision=lax.Precision.HIGHEST)
        + b_ref[...]
    )


def _embed_h(x, w, b, blk):
    n, d = x.shape
    return pl.pallas_call(
        _mm_bias_body,
        grid=(n // blk,),
        in_specs=[
            pl.BlockSpec((blk, d), lambda i: (i, _zi())),
            pl.BlockSpec((d, d), lambda i: (_zi(), _zi())),
            pl.BlockSpec((1, d), lambda i: (_zi(), _zi())),
        ],
        out_specs=pl.BlockSpec((blk, d), lambda i: (i, _zi())),
        out_shape=jax.ShapeDtypeStruct((n, d), F32),
    )(x, w, b)


def _embed_e_body(x_ref, w_ref, b_ref, o_ref):
    o_ref[...] = x_ref[...] * w_ref[...] + b_ref[...]


def _embed_e(x, w, b, blk):
    e, _ = x.shape
    d = w.shape[1]
    return pl.pallas_call(
        _embed_e_body,
        grid=(e // blk,),
        in_specs=[
            pl.BlockSpec((blk, 1), lambda i: (i, _zi())),
            pl.BlockSpec((1, d), lambda i: (_zi(), _zi())),
            pl.BlockSpec((1, d), lambda i: (_zi(), _zi())),
        ],
        out_specs=pl.BlockSpec((blk, d), lambda i: (i, _zi())),
        out_shape=jax.ShapeDtypeStruct((e, d), F32),
    )(x, w, b)


def _node_mm_body(h_ref, aw, ab, bw, bb, dw, db, ew, eb,
                  ah_o, t1_o, ehf_o):
    h = h_ref[...]
    d = h.shape[1]
    hh = d // 2

    def mm(w, b):
        return (jnp.dot(h, w[...], preferred_element_type=F32,
                        precision=lax.Precision.HIGHEST) + b[...])

    ah_o[...] = mm(aw, ab)
    bh = mm(bw, bb)
    dh = mm(dw, db)
    t1_o[0] = jnp.concatenate([dh[:, :hh], bh[:, :hh]], axis=1)
    t1_o[1] = jnp.concatenate([dh[:, hh:], bh[:, hh:]], axis=1)
    ehf_o[...] = mm(ew, eb)


def _node_mm(h, aw, ab, bw, bb, dw, db, ew, eb, blk):
    n, d = h.shape
    wspec = pl.BlockSpec((d, d), lambda i: (_zi(), _zi()))
    bspec = pl.BlockSpec((1, d), lambda i: (_zi(), _zi()))
    full = pl.BlockSpec((blk, d), lambda i: (i, _zi()))
    return pl.pallas_call(
        _node_mm_body,
        grid=(n // blk,),
        in_specs=[full, wspec, bspec, wspec, bspec, wspec, bspec, wspec,
                  bspec],
        out_specs=[full,
                   pl.BlockSpec((2, blk, d), lambda i: (_zi(), i, _zi())),
                   full],
        out_shape=[
            jax.ShapeDtypeStruct((n, d), F32),
            jax.ShapeDtypeStruct((2, n, d), F32),
            jax.ShapeDtypeStruct((n, d), F32),
        ],
    )(h, aw, ab, bw, bb, dw, db, ew, eb)


def _ce_mm_body(e_ref, w_ref, b_ref, c_o):
    ce = (jnp.dot(e_ref[...], w_ref[...], preferred_element_type=F32,
                  precision=lax.Precision.HIGHEST) + b_ref[...])
    hh = ce.shape[1] // 2
    c_o[0] = ce[:, :hh]
    c_o[1] = ce[:, hh:]


def _ce_mm(e, w, b, blk):
    n, d = e.shape
    hh = d // 2
    return pl.pallas_call(
        _ce_mm_body,
        grid=(n // blk,),
        in_specs=[
            pl.BlockSpec((blk, d), lambda i: (i, _zi())),
            pl.BlockSpec((d, d), lambda i: (_zi(), _zi())),
            pl.BlockSpec((1, d), lambda i: (_zi(), _zi())),
        ],
        out_specs=pl.BlockSpec((2, blk, hh), lambda i: (_zi(), i, _zi())),
        out_shape=jax.ShapeDtypeStruct((2, n, hh), F32),
    )(e, w, b)


def _node_ep_body(ah, num_p0, num_p1, den_p0, den_p1, hin, nnorm, stats,
                  bhg, bhb, beg, beb, ces,
                  h_o, esc_o, esh_o, *, n_edges):
    d = ah.shape[1]
    hh = d // 2
    num = jnp.concatenate([num_p0[0], num_p1[0]], axis=1)
    den = jnp.concatenate([den_p0[0], den_p1[0]], axis=1)
    hn = ah[...] + num / (den + 1e-6)
    hn = hn * nnorm[...]
    mu = jnp.mean(hn, axis=0, keepdims=True)
    var = jnp.mean((hn - mu) ** 2, axis=0, keepdims=True)
    hbn = (hn - mu) / jnp.sqrt(var + 1e-5) * bhg[...] + bhb[...]
    h_o[...] = hin[...] + jnp.maximum(hbn, 0.0)

    st = stats[...][:, 0, :]
    nsub = st.shape[0] // 2
    t0 = jnp.sum(st[:nsub], axis=0, keepdims=True)
    t1 = jnp.sum(st[nsub:], axis=0, keepdims=True)
    ssum = jnp.concatenate([t0[:, :hh], t1[:, :hh]], axis=1)
    ssq = jnp.concatenate([t0[:, hh:], t1[:, hh:]], axis=1)
    ce = ces[0, 0]
    m = ssum * (1.0 / n_edges)
    var_e = (ce * ce) * (ssq * (1.0 / n_edges) - m * m)
    inv = 1.0 / jnp.sqrt(var_e + 1e-5)
    esc_o[...] = (ce * inv) * beg[...]
    esh_o[...] = beb[...] - (ce * m) * inv * beg[...]


def _node_ep(ah, nump, denp, hin, nnorm, stats,
             bhg, bhb, beg, beb, ces, n_edges):
    n, d = ah.shape
    hh = d // 2
    nd_spec = pl.BlockSpec((n, d), lambda i: (_zi(), _zi()))
    row = pl.BlockSpec((1, d), lambda i: (_zi(), _zi()))

    def plane(p):
        return pl.BlockSpec((1, n, hh),
                            lambda i, p=p: (jnp.int32(p), _zi(), _zi()))

    return pl.pallas_call(
        functools.partial(_node_ep_body, n_edges=n_edges),
        grid=(1,),
        in_specs=[
            nd_spec, plane(0), plane(1), plane(0), plane(1), nd_spec,
            pl.BlockSpec((n, 1), lambda i: (_zi(), _zi())),
            pl.BlockSpec(stats.shape, lambda i: (_zi(), _zi(), _zi())),
            row, row, row, row,
            pl.BlockSpec((1, 1), lambda i: (_zi(), _zi())),
        ],
        out_specs=[nd_spec, row, row],
        out_shape=[jax.ShapeDtypeStruct((n, d), F32),
                   jax.ShapeDtypeStruct((1, d), F32),
                   jax.ShapeDtypeStruct((1, d), F32)],
    )(ah, nump, nump, denp, denp, hin, nnorm, stats,
      bhg, bhb, beg, beb, ces)


def _edge_ep_body(ein, en_p0, en_p1, esc, esh, e_o):
    en = jnp.concatenate([en_p0[0], en_p1[0]], axis=1)
    e_o[...] = ein[...] + jnp.maximum(en * esc[...] + esh[...], 0.0)


def _edge_ep(ein, enp, esc, esh, blk):
    e, d = ein.shape
    hh = d // 2
    full = pl.BlockSpec((blk, d), lambda i: (i, _zi()))
    row = pl.BlockSpec((1, d), lambda i: (_zi(), _zi()))

    def plane(p):
        return pl.BlockSpec((1, blk, hh),
                            lambda i, p=p: (jnp.int32(p), i, _zi()))

    return pl.pallas_call(
        _edge_ep_body,
        grid=(e // blk,),
        in_specs=[full, plane(0), plane(1), row, row],
        out_specs=full,
        out_shape=jax.ShapeDtypeStruct((e, d), F32),
    )(ein, enp, enp, esc, esh)


def _readout_body(h_ref, w_ref, o_ref, *, n_nodes):
    hm = jnp.sum(h_ref[...], axis=0, keepdims=True) * (1.0 / n_nodes)
    o_ref[...] = jnp.dot(hm, w_ref[...], preferred_element_type=F32,
                         precision=lax.Precision.HIGHEST)


def _readout(h, w):
    n, d = h.shape
    return pl.pallas_call(
        functools.partial(_readout_body, n_nodes=n),
        grid=(),
        in_specs=[pl.BlockSpec((n, d), lambda: (_zi(), _zi())),
                  pl.BlockSpec((d, d), lambda: (_zi(), _zi()))],
        out_specs=pl.BlockSpec((1, d), lambda: (_zi(), _zi())),
        out_shape=jax.ShapeDtypeStruct((1, d), F32),
    )(h, w)


_CHUNK = 64


def _sc_edge_body(n_nodes, n_pad, n_edges,
                  src, dst, cep, t1f, ehf,
                  enewp, nump, denp, stats,
                  srcv, srcv2, dstv, dstv2, dbv, ehv, cev, envv, sbuf,
                  statsv, acc, sem):
    c = lax.axis_index("c")
    s = lax.axis_index("s")
    hh = cev.shape[1]
    nchunk = n_edges // _CHUNK
    rows_per_tile = n_pad // 2 // 16
    cn = c * n_nodes

    zvec = jnp.zeros((16,), F32)

    base_r = s * rows_per_tile

    def zero_acc():
        def zb(r, t):
            for f in range(8):
                sbuf[r, pl.ds(f * 16, 16)] = zvec
            return t
        lax.fori_loop(jnp.int32(0), jnp.int32(_CHUNK), zb, jnp.int32(0))
        done = 0
        while done < rows_per_tile:
            sz = min(_CHUNK, rows_per_tile - done)
            pltpu.sync_copy(sbuf.at[pl.ds(0, sz)],
                            acc.at[pl.ds(base_r + done, sz)])
            done += sz

    def drain_acc(out):
        done = 0
        while done < rows_per_tile:
            sz = min(_CHUNK, rows_per_tile - done)
            r0 = base_r + done
            pltpu.sync_copy(acc.at[pl.ds(r0, sz)], sbuf.at[pl.ds(0, sz)])
            pltpu.sync_copy(sbuf.at[pl.ds(0, sz)], out.at[c, pl.ds(r0, sz)])
            done += sz

    zero_acc()
    plsc.subcore_barrier()

    nchunks_s = (jnp.int32(nchunk) - s + 15) // 16

    def chunk_body(i, st):
        g = s + i * 16
        base = g * _CHUNK
        pltpu.sync_copy(src.at[pl.ds(base, _CHUNK)], srcv)
        pltpu.sync_copy(dst.at[pl.ds(base, _CHUNK)], dstv.at[pl.ds(0, _CHUNK)])
        for k in range(_CHUNK // 16):
            sl = pl.ds(k * 16, 16)
            srcv2[sl] = srcv[sl] + cn
            dstv2[sl] = lax.shift_right_logical(dstv[sl], jnp.int32(1))
        ga = pltpu.async_copy(t1f.at[srcv2], dbv, sem)
        gb = pltpu.async_copy(ehf.at[dstv.at[pl.ds(0, _CHUNK)]], ehv, sem)
        pltpu.sync_copy(cep.at[c, pl.ds(base, _CHUNK)], cev)
        ga.wait()
        gb.wait()

        ehbase = c * hh

        def row_body(r, carry):
            sums = list(carry[:4])
            sqs = list(carry[4:])
            pv = dstv[pl.ds(r, 16)]
            coff = (pv[0] & 1) * hh
            opp = hh - coff
            for f in range(4):
                sl = pl.ds(f * 16, 16)
                en = (dbv[r, sl] + ehv[r, pl.ds(ehbase + f * 16, 16)]
                      + cev[r, sl])
                envv[r, sl] = en
                sg = 1.0 / (1.0 + jnp.exp(-en))
                sbuf[r, pl.ds(coff + f * 16, 16)] = (
                    sg * dbv[r, pl.ds(hh + f * 16, 16)])
                sbuf[r, pl.ds(opp + f * 16, 16)] = zvec
                sums[f] = sums[f] + en
                sqs[f] = sqs[f] + en * en
            return tuple(sums) + tuple(sqs)

        st = lax.fori_loop(jnp.int32(0), jnp.int32(_CHUNK), row_body, st)

        pltpu.sync_copy(sbuf, acc.at[dstv2], add=True)
        pltpu.sync_copy(envv, enewp.at[c, pl.ds(base, _CHUNK)])
        return st

    init = (zvec,) * 8
    st = lax.fori_loop(jnp.int32(0), nchunks_s, chunk_body, init)

    for f in range(4):
        statsv[0, 0, pl.ds(f * 16, 16)] = st[f]
        statsv[0, 0, pl.ds(hh + f * 16, 16)] = st[4 + f]
    wid = c * 16 + s
    pltpu.sync_copy(statsv, stats.at[pl.ds(wid, 1)])

    plsc.subcore_barrier()
    drain_acc(nump)
    plsc.subcore_barrier()
    zero_acc()
    plsc.subcore_barrier()

    def chunk_body2(i, t):
        g = s + i * 16
        base = g * _CHUNK
        pltpu.sync_copy(dst.at[pl.ds(base, _CHUNK)], dstv.at[pl.ds(0, _CHUNK)])
        pltpu.sync_copy(enewp.at[c, pl.ds(base, _CHUNK)], envv)
        for k in range(_CHUNK // 16):
            sl = pl.ds(k * 16, 16)
            dstv2[sl] = lax.shift_right_logical(dstv[sl], jnp.int32(1))

        def row_body2(r, t2):
            pv = dstv[pl.ds(r, 16)]
            coff = (pv[0] & 1) * hh
            opp = hh - coff
            for f in range(4):
                sl = pl.ds(f * 16, 16)
                sbuf[r, pl.ds(coff + f * 16, 16)] = (
                    1.0 / (1.0 + jnp.exp(-envv[r, pl.ds(f * 16, 16)])))
                sbuf[r, pl.ds(opp + f * 16, 16)] = zvec
            return t2

        lax.fori_loop(jnp.int32(0), jnp.int32(_CHUNK), row_body2,
                      jnp.int32(0))
        pltpu.sync_copy(sbuf, acc.at[dstv2], add=True)
        return t

    lax.fori_loop(jnp.int32(0), nchunks_s, chunk_body2, jnp.int32(0))

    plsc.subcore_barrier()
    drain_acc(denp)


def _sc_edge(src, dst, cep, t1f, ehf):
    n_nodes = ehf.shape[0]
    n_edges = src.shape[0]
    d = ehf.shape[1]
    hh = d // 2
    n_pad = ((n_nodes + 255) // 256) * 256
    mesh = plsc.VectorSubcoreMesh(core_axis_name="c", subcore_axis_name="s")
    fn = pl.kernel(
        functools.partial(_sc_edge_body, n_nodes, n_pad, n_edges),
        out_type=[
            jax.ShapeDtypeStruct((2, n_edges, hh), F32),
            jax.ShapeDtypeStruct((2, n_pad // 2, d), F32),
            jax.ShapeDtypeStruct((2, n_pad // 2, d), F32),
            jax.ShapeDtypeStruct((32, 1, d), F32),
        ],
        mesh=mesh,
        scratch_types=[
            pltpu.VMEM((_CHUNK,), jnp.int32),
            pltpu.VMEM((_CHUNK,), jnp.int32),
            pltpu.VMEM((_CHUNK + 16,), jnp.int32),
            pltpu.VMEM((_CHUNK,), jnp.int32),
            pltpu.VMEM((_CHUNK, d), F32),
            pltpu.VMEM((_CHUNK, d), F32),
            pltpu.VMEM((_CHUNK, hh), F32),
            pltpu.VMEM((_CHUNK, hh), F32),
            pltpu.VMEM((_CHUNK, d), F32),
            pltpu.VMEM((1, 1, d), F32),
            pltpu.VMEM_SHARED((n_pad // 2, d), F32),
            pltpu.SemaphoreType.DMA,
        ],
    )
    return fn(src, dst, cep, t1f, ehf)


def kernel(edge_index, nodes_feat, edges_feat, nodes_num_norm_sqrt,
           edges_num_norm_sqrt, emb_h_w, emb_h_b, emb_e_w, emb_e_b,
           Aw, Ab, Bw, Bb, Cw, Cb, Dw, Db, Ew, Eb,
           bn_h_g, bn_h_b, bn_e_g, bn_e_b, readout_w):
    n, d = nodes_feat.shape
    n_edges = edge_index.shape[1]
    num_layers = Aw.shape[0]

    src = edge_index[0].astype(jnp.int32)
    dst = edge_index[1].astype(jnp.int32)
    ces = edges_num_norm_sqrt[0:1, 0:1].astype(F32)
    nnorm = nodes_num_norm_sqrt.astype(F32)

    nblk = 2000
    eblk = 4000

    h = _embed_h(nodes_feat.astype(F32), emb_h_w.astype(F32),
                 emb_h_b.reshape(1, d).astype(F32), nblk)
    e = _embed_e(edges_feat.astype(F32), emb_e_w.astype(F32),
                 emb_e_b.reshape(1, d).astype(F32), eblk)

    for l in range(num_layers):
        ah, t1, ehf = _node_mm(
            h, Aw[l], Ab[l].reshape(1, d), Bw[l], Bb[l].reshape(1, d),
            Dw[l], Db[l].reshape(1, d), Ew[l], Eb[l].reshape(1, d), nblk)
        cep = _ce_mm(e, Cw[l], Cb[l].reshape(1, d), eblk)
        t1f = t1.reshape(2 * n, d)
        enp, nump, denp, stats = _sc_edge(src, dst, cep, t1f, ehf)
        hh = d // 2
        n_pad2 = nump.shape[1]
        nump = nump.reshape(2, 2 * n_pad2, hh)
        denp = denp.reshape(2, 2 * n_pad2, hh)
        h, esc, esh = _node_ep(
            ah, nump, denp, h, nnorm, stats,
            bn_h_g[l].reshape(1, d), bn_h_b[l].reshape(1, d),
            bn_e_g[l].reshape(1, d), bn_e_b[l].reshape(1, d), ces, n_edges)
        e = _edge_ep(e, enp, esc, esh, eblk)

    return _readout(h, readout_w.astype(F32))


# speedup vs baseline: 1.6567x; 1.6567x over previous
import functools

import jax
import jax.numpy as jnp
from jax import lax
from jax.experimental import pallas as pl
from jax.experimental.pallas import tpu as pltpu
from jax.experimental.pallas import tpu_sc as plsc

F32 = jnp.float32


def _zi():
    return jnp.int32(0)


def _mm_bias_body(x_ref, w_ref, b_ref, o_ref):
    o_ref[...] = (
        jnp.dot(x_ref[...], w_ref[...], preferred_element_type=F32,
                precision=lax.Precision.HIGHEST)
        + b_ref[...]
    )


def _embed_h(x, w, b, blk):
    n, d = x.shape
    return pl.pallas_call(
        _mm_bias_body,
        grid=(n // blk,),
        in_specs=[
            pl.BlockSpec((blk, d), lambda i: (i, _zi())),
            pl.BlockSpec((d, d), lambda i: (_zi(), _zi())),
            pl.BlockSpec((1, d), lambda i: (_zi(), _zi())),
        ],
        out_specs=pl.BlockSpec((blk, d), lambda i: (i, _zi())),
        out_shape=jax.ShapeDtypeStruct((n, d), F32),
    )(x, w, b)


def _embed_e_body(x_ref, w_ref, b_ref, o_ref):
    o_ref[...] = x_ref[...] * w_ref[...] + b_ref[...]


def _embed_e(x, w, b, blk):
    e, _ = x.shape
    d = w.shape[1]
    return pl.pallas_call(
        _embed_e_body,
        grid=(e // blk,),
        in_specs=[
            pl.BlockSpec((blk, 1), lambda i: (i, _zi())),
            pl.BlockSpec((1, d), lambda i: (_zi(), _zi())),
            pl.BlockSpec((1, d), lambda i: (_zi(), _zi())),
        ],
        out_specs=pl.BlockSpec((blk, d), lambda i: (i, _zi())),
        out_shape=jax.ShapeDtypeStruct((e, d), F32),
    )(x, w, b)


def _node_mm_body(h_ref, aw, ab, bw, bb, dw, db, ew, eb,
                  ah_o, t1_o, ehf_o):
    h = h_ref[...]
    d = h.shape[1]
    hh = d // 2

    def mm(w, b):
        return (jnp.dot(h, w[...], preferred_element_type=F32,
                        precision=lax.Precision.HIGHEST) + b[...])

    ah_o[...] = mm(aw, ab)
    bh = mm(bw, bb)
    dh = mm(dw, db)
    t1_o[0] = jnp.concatenate([dh[:, :hh], bh[:, :hh]], axis=1)
    t1_o[1] = jnp.concatenate([dh[:, hh:], bh[:, hh:]], axis=1)
    ehf_o[...] = mm(ew, eb)


def _node_mm(h, aw, ab, bw, bb, dw, db, ew, eb, blk):
    n, d = h.shape
    wspec = pl.BlockSpec((d, d), lambda i: (_zi(), _zi()))
    bspec = pl.BlockSpec((1, d), lambda i: (_zi(), _zi()))
    full = pl.BlockSpec((blk, d), lambda i: (i, _zi()))
    return pl.pallas_call(
        _node_mm_body,
        grid=(n // blk,),
        in_specs=[full, wspec, bspec, wspec, bspec, wspec, bspec, wspec,
                  bspec],
        out_specs=[full,
                   pl.BlockSpec((2, blk, d), lambda i: (_zi(), i, _zi())),
                   full],
        out_shape=[
            jax.ShapeDtypeStruct((n, d), F32),
            jax.ShapeDtypeStruct((2, n, d), F32),
            jax.ShapeDtypeStruct((n, d), F32),
        ],
    )(h, aw, ab, bw, bb, dw, db, ew, eb)


def _ce_mm_body(e_ref, w_ref, b_ref, c_o):
    ce = (jnp.dot(e_ref[...], w_ref[...], preferred_element_type=F32,
                  precision=lax.Precision.HIGHEST) + b_ref[...])
    hh = ce.shape[1] // 2
    c_o[0] = ce[:, :hh]
    c_o[1] = ce[:, hh:]


def _ce_mm(e, w, b, blk):
    n, d = e.shape
    hh = d // 2
    return pl.pallas_call(
        _ce_mm_body,
        grid=(n // blk,),
        in_specs=[
            pl.BlockSpec((blk, d), lambda i: (i, _zi())),
            pl.BlockSpec((d, d), lambda i: (_zi(), _zi())),
            pl.BlockSpec((1, d), lambda i: (_zi(), _zi())),
        ],
        out_specs=pl.BlockSpec((2, blk, hh), lambda i: (_zi(), i, _zi())),
        out_shape=jax.ShapeDtypeStruct((2, n, hh), F32),
    )(e, w, b)


def _node_ep_body(ah, nd_p0, nd_p1, hin, nnorm, stats,
                  bhg, bhb, beg, beb, ces,
                  h_o, esc_o, esh_o, *, n_edges):
    d = ah.shape[1]
    hh = d // 2
    num = jnp.concatenate([nd_p0[0][:, :hh], nd_p1[0][:, :hh]], axis=1)
    den = jnp.concatenate([nd_p0[0][:, hh:], nd_p1[0][:, hh:]], axis=1)
    hn = ah[...] + num / (den + 1e-6)
    hn = hn * nnorm[...]
    mu = jnp.mean(hn, axis=0, keepdims=True)
    var = jnp.mean((hn - mu) ** 2, axis=0, keepdims=True)
    hbn = (hn - mu) / jnp.sqrt(var + 1e-5) * bhg[...] + bhb[...]
    h_o[...] = hin[...] + jnp.maximum(hbn, 0.0)

    st = stats[...][:, 0, :]
    nsub = st.shape[0] // 2
    t0 = jnp.sum(st[:nsub], axis=0, keepdims=True)
    t1 = jnp.sum(st[nsub:], axis=0, keepdims=True)
    ssum = jnp.concatenate([t0[:, :hh], t1[:, :hh]], axis=1)
    ssq = jnp.concatenate([t0[:, hh:], t1[:, hh:]], axis=1)
    ce = ces[0, 0]
    m = ssum * (1.0 / n_edges)
    var_e = (ce * ce) * (ssq * (1.0 / n_edges) - m * m)
    inv = 1.0 / jnp.sqrt(var_e + 1e-5)
    esc_o[...] = (ce * inv) * beg[...]
    esh_o[...] = beb[...] - (ce * m) * inv * beg[...]


def _node_ep(ah, ndp, hin, nnorm, stats,
             bhg, bhb, beg, beb, ces, n_edges):
    n, d = ah.shape
    nd_spec = pl.BlockSpec((n, d), lambda i: (_zi(), _zi()))
    row = pl.BlockSpec((1, d), lambda i: (_zi(), _zi()))

    def plane(p):
        return pl.BlockSpec((1, n, d),
                            lambda i, p=p: (jnp.int32(p), _zi(), _zi()))

    return pl.pallas_call(
        functools.partial(_node_ep_body, n_edges=n_edges),
        grid=(1,),
        in_specs=[
            nd_spec, plane(0), plane(1), nd_spec,
            pl.BlockSpec((n, 1), lambda i: (_zi(), _zi())),
            pl.BlockSpec(stats.shape, lambda i: (_zi(), _zi(), _zi())),
            row, row, row, row,
            pl.BlockSpec((1, 1), lambda i: (_zi(), _zi())),
        ],
        out_specs=[nd_spec, row, row],
        out_shape=[jax.ShapeDtypeStruct((n, d), F32),
                   jax.ShapeDtypeStruct((1, d), F32),
                   jax.ShapeDtypeStruct((1, d), F32)],
    )(ah, ndp, ndp, hin, nnorm, stats,
      bhg, bhb, beg, beb, ces)


def _edge_ep_body(ein, en_p0, en_p1, esc, esh, e_o):
    en = jnp.concatenate([en_p0[0], en_p1[0]], axis=1)
    e_o[...] = ein[...] + jnp.maximum(en * esc[...] + esh[...], 0.0)


def _edge_ep(ein, enp, esc, esh, blk):
    e, d = ein.shape
    hh = d // 2
    full = pl.BlockSpec((blk, d), lambda i: (i, _zi()))
    row = pl.BlockSpec((1, d), lambda i: (_zi(), _zi()))

    def plane(p):
        return pl.BlockSpec((1, blk, hh),
                            lambda i, p=p: (jnp.int32(p), i, _zi()))

    return pl.pallas_call(
        _edge_ep_body,
        grid=(e // blk,),
        in_specs=[full, plane(0), plane(1), row, row],
        out_specs=full,
        out_shape=jax.ShapeDtypeStruct((e, d), F32),
    )(ein, enp, enp, esc, esh)


def _readout_body(h_ref, w_ref, o_ref, *, n_nodes):
    hm = jnp.sum(h_ref[...], axis=0, keepdims=True) * (1.0 / n_nodes)
    o_ref[...] = jnp.dot(hm, w_ref[...], preferred_element_type=F32,
                         precision=lax.Precision.HIGHEST)


def _readout(h, w):
    n, d = h.shape
    return pl.pallas_call(
        functools.partial(_readout_body, n_nodes=n),
        grid=(),
        in_specs=[pl.BlockSpec((n, d), lambda: (_zi(), _zi())),
                  pl.BlockSpec((d, d), lambda: (_zi(), _zi()))],
        out_specs=pl.BlockSpec((1, d), lambda: (_zi(), _zi())),
        out_shape=jax.ShapeDtypeStruct((1, d), F32),
    )(h, w)


_CHUNK = 64


def _sc_edge_body(n_nodes, n_pad, n_edges,
                  src, dst, cep, t1f, ehf,
                  enewp, ndp, stats,
                  srcv, srcv2, dstv, dbv, ehv, cev,
                  statsv, acc, sem):
    c = lax.axis_index("c")
    s = lax.axis_index("s")
    hh = cev.shape[1]
    nchunk = n_edges // _CHUNK
    rows_per_tile = n_pad // 16
    cn = c * n_nodes

    zvec = jnp.zeros((16,), F32)
    base_r = s * rows_per_tile

    def zb(r, t):
        for f in range(8):
            dbv[r, pl.ds(f * 16, 16)] = zvec
        return t
    lax.fori_loop(jnp.int32(0), jnp.int32(_CHUNK), zb, jnp.int32(0))
    done = 0
    while done < rows_per_tile:
        sz = min(_CHUNK, rows_per_tile - done)
        pltpu.sync_copy(dbv.at[pl.ds(0, sz)],
                        acc.at[pl.ds(base_r + done, sz)])
        done += sz
    plsc.subcore_barrier()

    nchunks_s = (jnp.int32(nchunk) - s + 15) // 16

    def chunk_body(i, st):
        g = s + i * 16
        base = g * _CHUNK
        pltpu.sync_copy(src.at[pl.ds(base, _CHUNK)], srcv)
        pltpu.sync_copy(dst.at[pl.ds(base, _CHUNK)], dstv)
        for k in range(_CHUNK // 16):
            sl = pl.ds(k * 16, 16)
            srcv2[sl] = srcv[sl] + cn
        ga = pltpu.async_copy(t1f.at[srcv2], dbv, sem)
        gb = pltpu.async_copy(ehf.at[dstv], ehv, sem)
        pltpu.sync_copy(cep.at[c, pl.ds(base, _CHUNK)], cev)
        ga.wait()
        gb.wait()

        ehbase = c * hh

        def row_body(r, carry):
            sums = list(carry[:4])
            sqs = list(carry[4:])
            for f in range(4):
                sl = pl.ds(f * 16, 16)
                en = (dbv[r, sl] + ehv[r, pl.ds(ehbase + f * 16, 16)]
                      + cev[r, sl])
                cev[r, sl] = en
                sg = 1.0 / (1.0 + jnp.exp(-en))
                dbv[r, sl] = sg * dbv[r, pl.ds(hh + f * 16, 16)]
                dbv[r, pl.ds(hh + f * 16, 16)] = sg
                sums[f] = sums[f] + en
                sqs[f] = sqs[f] + en * en
            return tuple(sums) + tuple(sqs)

        st = lax.fori_loop(jnp.int32(0), jnp.int32(_CHUNK), row_body, st)

        pltpu.sync_copy(dbv, acc.at[dstv], add=True)
        pltpu.sync_copy(cev, enewp.at[c, pl.ds(base, _CHUNK)])
        return st

    init = (zvec,) * 8
    st = lax.fori_loop(jnp.int32(0), nchunks_s, chunk_body, init)

    for f in range(4):
        statsv[0, 0, pl.ds(f * 16, 16)] = st[f]
        statsv[0, 0, pl.ds(hh + f * 16, 16)] = st[4 + f]
    wid = c * 16 + s
    pltpu.sync_copy(statsv, stats.at[pl.ds(wid, 1)])

    plsc.subcore_barrier()
    done = 0
    while done < rows_per_tile:
        sz = min(_CHUNK, rows_per_tile - done)
        r0 = base_r + done
        pltpu.sync_copy(acc.at[pl.ds(r0, sz)], dbv.at[pl.ds(0, sz)])
        pltpu.sync_copy(dbv.at[pl.ds(0, sz)], ndp.at[c, pl.ds(r0, sz)])
        done += sz


def _sc_edge(src, dst, cep, t1f, ehf):
    n_nodes = ehf.shape[0]
    n_edges = src.shape[0]
    d = ehf.shape[1]
    hh = d // 2
    n_pad = ((n_nodes + 127) // 128) * 128
    mesh = plsc.VectorSubcoreMesh(core_axis_name="c", subcore_axis_name="s")
    fn = pl.kernel(
        functools.partial(_sc_edge_body, n_nodes, n_pad, n_edges),
        out_type=[
            jax.ShapeDtypeStruct((2, n_edges, hh), F32),
            jax.ShapeDtypeStruct((2, n_pad, d), F32),
            jax.ShapeDtypeStruct((32, 1, d), F32),
        ],
        mesh=mesh,
        scratch_types=[
            pltpu.VMEM((_CHUNK,), jnp.int32),
            pltpu.VMEM((_CHUNK,), jnp.int32),
            pltpu.VMEM((_CHUNK,), jnp.int32),
            pltpu.VMEM((_CHUNK, d), F32),
            pltpu.VMEM((_CHUNK, d), F32),
            pltpu.VMEM((_CHUNK, hh), F32),
            pltpu.VMEM((1, 1, d), F32),
            pltpu.VMEM_SHARED((n_pad, d), F32),
            pltpu.SemaphoreType.DMA,
        ],
    )
    return fn(src, dst, cep, t1f, ehf)


def kernel(edge_index, nodes_feat, edges_feat, nodes_num_norm_sqrt,
           edges_num_norm_sqrt, emb_h_w, emb_h_b, emb_e_w, emb_e_b,
           Aw, Ab, Bw, Bb, Cw, Cb, Dw, Db, Ew, Eb,
           bn_h_g, bn_h_b, bn_e_g, bn_e_b, readout_w):
    n, d = nodes_feat.shape
    n_edges = edge_index.shape[1]
    num_layers = Aw.shape[0]

    src = edge_index[0].astype(jnp.int32)
    dst = edge_index[1].astype(jnp.int32)
    ces = edges_num_norm_sqrt[0:1, 0:1].astype(F32)
    nnorm = nodes_num_norm_sqrt.astype(F32)

    nblk = 2000
    eblk = 4000

    h = _embed_h(nodes_feat.astype(F32), emb_h_w.astype(F32),
                 emb_h_b.reshape(1, d).astype(F32), nblk)
    e = _embed_e(edges_feat.astype(F32), emb_e_w.astype(F32),
                 emb_e_b.reshape(1, d).astype(F32), eblk)

    for l in range(num_layers):
        ah, t1, ehf = _node_mm(
            h, Aw[l], Ab[l].reshape(1, d), Bw[l], Bb[l].reshape(1, d),
            Dw[l], Db[l].reshape(1, d), Ew[l], Eb[l].reshape(1, d), nblk)
        cep = _ce_mm(e, Cw[l], Cb[l].reshape(1, d), eblk)
        t1f = t1.reshape(2 * n, d)
        enp, ndp, stats = _sc_edge(src, dst, cep, t1f, ehf)
        h, esc, esh = _node_ep(
            ah, ndp, h, nnorm, stats,
            bn_h_g[l].reshape(1, d), bn_h_b[l].reshape(1, d),
            bn_e_g[l].reshape(1, d), bn_e_b[l].reshape(1, d), ces, n_edges)
        e = _edge_ep(e, enp, esc, esh, eblk)

    return _readout(h, readout_w.astype(F32))


# speedup vs baseline: 2.0183x; 1.2183x over previous
import functools

import jax
import jax.numpy as jnp
from jax import lax
from jax.experimental import pallas as pl
from jax.experimental.pallas import tpu as pltpu
from jax.experimental.pallas import tpu_sc as plsc

F32 = jnp.float32


def _zi():
    return jnp.int32(0)


def _mm_bias_body(x_ref, w_ref, b_ref, o_ref):
    o_ref[...] = (
        jnp.dot(x_ref[...], w_ref[...], preferred_element_type=F32,
                precision=lax.Precision.HIGHEST)
        + b_ref[...]
    )


def _embed_h(x, w, b, blk):
    n, d = x.shape
    return pl.pallas_call(
        _mm_bias_body,
        grid=(n // blk,),
        in_specs=[
            pl.BlockSpec((blk, d), lambda i: (i, _zi())),
            pl.BlockSpec((d, d), lambda i: (_zi(), _zi())),
            pl.BlockSpec((1, d), lambda i: (_zi(), _zi())),
        ],
        out_specs=pl.BlockSpec((blk, d), lambda i: (i, _zi())),
        out_shape=jax.ShapeDtypeStruct((n, d), F32),
    )(x, w, b)


def _embed_e_body(x_ref, w_ref, b_ref, o_ref):
    o_ref[...] = x_ref[...] * w_ref[...] + b_ref[...]


def _embed_e(x, w, b, blk):
    e, _ = x.shape
    d = w.shape[1]
    return pl.pallas_call(
        _embed_e_body,
        grid=(e // blk,),
        in_specs=[
            pl.BlockSpec((blk, 1), lambda i: (i, _zi())),
            pl.BlockSpec((1, d), lambda i: (_zi(), _zi())),
            pl.BlockSpec((1, d), lambda i: (_zi(), _zi())),
        ],
        out_specs=pl.BlockSpec((blk, d), lambda i: (i, _zi())),
        out_shape=jax.ShapeDtypeStruct((e, d), F32),
    )(x, w, b)


def _node_mm_body(h_ref, aw, ab, bw, bb, dw, db, ew, eb,
                  ah_o, t1_o, ehf_o):
    h = h_ref[...]
    d = h.shape[1]
    hh = d // 2

    def mm(w, b):
        return (jnp.dot(h, w[...], preferred_element_type=F32,
                        precision=lax.Precision.HIGHEST) + b[...])

    ah_o[...] = mm(aw, ab)
    bh = mm(bw, bb)
    dh = mm(dw, db)
    t1_o[0] = jnp.concatenate([dh[:, :hh], bh[:, :hh]], axis=1)
    t1_o[1] = jnp.concatenate([dh[:, hh:], bh[:, hh:]], axis=1)
    ehf_o[...] = mm(ew, eb)


def _node_mm(h, aw, ab, bw, bb, dw, db, ew, eb, blk):
    n, d = h.shape
    wspec = pl.BlockSpec((d, d), lambda i: (_zi(), _zi()))
    bspec = pl.BlockSpec((1, d), lambda i: (_zi(), _zi()))
    full = pl.BlockSpec((blk, d), lambda i: (i, _zi()))
    return pl.pallas_call(
        _node_mm_body,
        grid=(n // blk,),
        in_specs=[full, wspec, bspec, wspec, bspec, wspec, bspec, wspec,
                  bspec],
        out_specs=[full,
                   pl.BlockSpec((2, blk, d), lambda i: (_zi(), i, _zi())),
                   full],
        out_shape=[
            jax.ShapeDtypeStruct((n, d), F32),
            jax.ShapeDtypeStruct((2, n, d), F32),
            jax.ShapeDtypeStruct((n, d), F32),
        ],
    )(h, aw, ab, bw, bb, dw, db, ew, eb)


def _ce_mm_body(e_ref, w_ref, b_ref, c_o):
    ce = (jnp.dot(e_ref[...], w_ref[...], preferred_element_type=F32,
                  precision=lax.Precision.HIGHEST) + b_ref[...])
    hh = ce.shape[1] // 2
    c_o[0] = ce[:, :hh]
    c_o[1] = ce[:, hh:]


def _ce_mm(e, w, b, blk):
    n, d = e.shape
    hh = d // 2
    return pl.pallas_call(
        _ce_mm_body,
        grid=(n // blk,),
        in_specs=[
            pl.BlockSpec((blk, d), lambda i: (i, _zi())),
            pl.BlockSpec((d, d), lambda i: (_zi(), _zi())),
            pl.BlockSpec((1, d), lambda i: (_zi(), _zi())),
        ],
        out_specs=pl.BlockSpec((2, blk, hh), lambda i: (_zi(), i, _zi())),
        out_shape=jax.ShapeDtypeStruct((2, n, hh), F32),
    )(e, w, b)


def _node_ep_body(ah, nd_p0, nd_p1, hin, nnorm, stats,
                  bhg, bhb, beg, beb, ces,
                  h_o, esc_o, esh_o, *, n_edges):
    d = ah.shape[1]
    hh = d // 2
    num = jnp.concatenate([nd_p0[0][:, :hh], nd_p1[0][:, :hh]], axis=1)
    den = jnp.concatenate([nd_p0[0][:, hh:], nd_p1[0][:, hh:]], axis=1)
    hn = ah[...] + num / (den + 1e-6)
    hn = hn * nnorm[...]
    mu = jnp.mean(hn, axis=0, keepdims=True)
    var = jnp.mean((hn - mu) ** 2, axis=0, keepdims=True)
    hbn = (hn - mu) / jnp.sqrt(var + 1e-5) * bhg[...] + bhb[...]
    h_o[...] = hin[...] + jnp.maximum(hbn, 0.0)

    st = stats[...][:, 0, :]
    nsub = st.shape[0] // 2
    t0 = jnp.sum(st[:nsub], axis=0, keepdims=True)
    t1 = jnp.sum(st[nsub:], axis=0, keepdims=True)
    ssum = jnp.concatenate([t0[:, :hh], t1[:, :hh]], axis=1)
    ssq = jnp.concatenate([t0[:, hh:], t1[:, hh:]], axis=1)
    ce = ces[0, 0]
    m = ssum * (1.0 / n_edges)
    var_e = (ce * ce) * (ssq * (1.0 / n_edges) - m * m)
    inv = 1.0 / jnp.sqrt(var_e + 1e-5)
    esc_o[...] = (ce * inv) * beg[...]
    esh_o[...] = beb[...] - (ce * m) * inv * beg[...]


def _node_ep(ah, ndp, hin, nnorm, stats,
             bhg, bhb, beg, beb, ces, n_edges):
    n, d = ah.shape
    nd_spec = pl.BlockSpec((n, d), lambda i: (_zi(), _zi()))
    row = pl.BlockSpec((1, d), lambda i: (_zi(), _zi()))

    def plane(p):
        return pl.BlockSpec((1, n, d),
                            lambda i, p=p: (jnp.int32(p), _zi(), _zi()))

    return pl.pallas_call(
        functools.partial(_node_ep_body, n_edges=n_edges),
        grid=(1,),
        in_specs=[
            nd_spec, plane(0), plane(1), nd_spec,
            pl.BlockSpec((n, 1), lambda i: (_zi(), _zi())),
            pl.BlockSpec(stats.shape, lambda i: (_zi(), _zi(), _zi())),
            row, row, row, row,
            pl.BlockSpec((1, 1), lambda i: (_zi(), _zi())),
        ],
        out_specs=[nd_spec, row, row],
        out_shape=[jax.ShapeDtypeStruct((n, d), F32),
                   jax.ShapeDtypeStruct((1, d), F32),
                   jax.ShapeDtypeStruct((1, d), F32)],
    )(ah, ndp, ndp, hin, nnorm, stats,
      bhg, bhb, beg, beb, ces)


def _edge_ep_body(ein, en_p0, en_p1, esc, esh, e_o):
    en = jnp.concatenate([en_p0[0], en_p1[0]], axis=1)
    e_o[...] = ein[...] + jnp.maximum(en * esc[...] + esh[...], 0.0)


def _edge_ep(ein, enp, esc, esh, blk):
    e, d = ein.shape
    hh = d // 2
    full = pl.BlockSpec((blk, d), lambda i: (i, _zi()))
    row = pl.BlockSpec((1, d), lambda i: (_zi(), _zi()))

    def plane(p):
        return pl.BlockSpec((1, blk, hh),
                            lambda i, p=p: (jnp.int32(p), i, _zi()))

    return pl.pallas_call(
        _edge_ep_body,
        grid=(e // blk,),
        in_specs=[full, plane(0), plane(1), row, row],
        out_specs=full,
        out_shape=jax.ShapeDtypeStruct((e, d), F32),
    )(ein, enp, enp, esc, esh)


def _readout_body(h_ref, w_ref, o_ref, *, n_nodes):
    hm = jnp.sum(h_ref[...], axis=0, keepdims=True) * (1.0 / n_nodes)
    o_ref[...] = jnp.dot(hm, w_ref[...], preferred_element_type=F32,
                         precision=lax.Precision.HIGHEST)


def _readout(h, w):
    n, d = h.shape
    return pl.pallas_call(
        functools.partial(_readout_body, n_nodes=n),
        grid=(),
        in_specs=[pl.BlockSpec((n, d), lambda: (_zi(), _zi())),
                  pl.BlockSpec((d, d), lambda: (_zi(), _zi()))],
        out_specs=pl.BlockSpec((1, d), lambda: (_zi(), _zi())),
        out_shape=jax.ShapeDtypeStruct((1, d), F32),
    )(h, w)


_CHUNK = 64


def _sc_edge_body(n_nodes, n_pad, n_edges,
                  src, dst, cep, t1f, ehf,
                  enewp, ndp, stats,
                  srcv0, srcv20, dstv0, dbv0, ehv0, cev0,
                  srcv1, srcv21, dstv1, dbv1, ehv1, cev1,
                  statsv, acc, semg0, semg1, semi0, semi1):
    c = lax.axis_index("c")
    s = lax.axis_index("s")
    hh = cev0.shape[1]
    nchunk = n_edges // _CHUNK
    rows_per_tile = n_pad // 16
    cn = c * n_nodes

    zvec = jnp.zeros((16,), F32)
    base_r = s * rows_per_tile

    def zb(r, t):
        for f in range(8):
            dbv0[r, pl.ds(f * 16, 16)] = zvec
        return t
    lax.fori_loop(jnp.int32(0), jnp.int32(_CHUNK), zb, jnp.int32(0))
    done = 0
    while done < rows_per_tile:
        sz = min(_CHUNK, rows_per_tile - done)
        pltpu.sync_copy(dbv0.at[pl.ds(0, sz)],
                        acc.at[pl.ds(base_r + done, sz)])
        done += sz
    plsc.subcore_barrier()

    per2 = (nchunk // 2) // 16
    rem2 = (nchunk // 2) % 16
    lt = (s < rem2).astype(jnp.int32)
    cnt = 2 * per2 + 2 * lt
    start = 2 * per2 * s + 2 * jnp.minimum(s, jnp.int32(rem2))
    npairs = per2 + lt

    def fetch_idx(k, sv, dv, sem):
        base = k * _CHUNK
        pltpu.async_copy(src.at[pl.ds(base, _CHUNK)], sv, sem)
        pltpu.async_copy(dst.at[pl.ds(base, _CHUNK)], dv, sem)

    def wait_idx(k, sv, dv, sem):
        base = k * _CHUNK
        pltpu.make_async_copy(src.at[pl.ds(base, _CHUNK)], sv, sem).wait()
        pltpu.make_async_copy(dst.at[pl.ds(base, _CHUNK)], dv, sem).wait()

    def issue_gathers(k, sv, sv2, dv, db, eh, ce, sem):
        base = k * _CHUNK
        for kk in range(_CHUNK // 16):
            sl = pl.ds(kk * 16, 16)
            sv2[sl] = sv[sl] + cn
        pltpu.async_copy(t1f.at[sv2], db, sem)
        pltpu.async_copy(ehf.at[dv], eh, sem)
        pltpu.async_copy(cep.at[c, pl.ds(base, _CHUNK)], ce, sem)

    ehbase = c * hh

    def make_row_body(db, eh, ce):
        def row_body(r, carry):
            sums = list(carry[:4])
            sqs = list(carry[4:])
            for f in range(4):
                sl = pl.ds(f * 16, 16)
                en = (db[r, sl] + eh[r, pl.ds(ehbase + f * 16, 16)]
                      + ce[r, sl])
                ce[r, sl] = en
                sg = 1.0 / (1.0 + jnp.exp(-en))
                db[r, sl] = sg * db[r, pl.ds(hh + f * 16, 16)]
                db[r, pl.ds(hh + f * 16, 16)] = sg
                sums[f] = sums[f] + en
                sqs[f] = sqs[f] + en * en
            return tuple(sums) + tuple(sqs)
        return row_body

    def process(k, st, bufb, bufo, semgb, semgo, semib,
                has_next, has_next2):
        sv_b, sv2_b, dv_b, db_b, eh_b, ce_b = bufb
        sv_o, sv2_o, dv_o, db_o, eh_o, ce_o = bufo
        base = k * _CHUNK
        pltpu.make_async_copy(t1f.at[sv2_b], db_b, semgb).wait()
        pltpu.make_async_copy(ehf.at[dv_b], eh_b, semgb).wait()
        pltpu.make_async_copy(cep.at[c, pl.ds(base, _CHUNK)], ce_b,
                              semgb).wait()
        if has_next is True:
            wait_idx(k + 1, sv_o, dv_o, semgo[1])
            issue_gathers(k + 1, sv_o, sv2_o, dv_o, db_o, eh_o, ce_o,
                          semgo[0])
        else:
            @pl.when(has_next)
            def _():
                wait_idx(k + 1, sv_o, dv_o, semgo[1])
                issue_gathers(k + 1, sv_o, sv2_o, dv_o, db_o, eh_o, ce_o,
                              semgo[0])
        st = lax.fori_loop(jnp.int32(0), jnp.int32(_CHUNK),
                           make_row_body(db_b, eh_b, ce_b), st)
        pltpu.sync_copy(db_b, acc.at[dv_b], add=True)
        pltpu.sync_copy(ce_b, enewp.at[c, pl.ds(base, _CHUNK)])

        @pl.when(has_next2)
        def _():
            fetch_idx(k + 2, sv_b, dv_b, semib)
        return st

    buf0 = (srcv0, srcv20, dstv0, dbv0, ehv0, cev0)
    buf1 = (srcv1, srcv21, dstv1, dbv1, ehv1, cev1)

    pltpu.sync_copy(src.at[pl.ds(start * _CHUNK, _CHUNK)], srcv0)
    pltpu.sync_copy(dst.at[pl.ds(start * _CHUNK, _CHUNK)], dstv0)
    issue_gathers(start, srcv0, srcv20, dstv0, dbv0, ehv0, cev0, semg0)
    fetch_idx(start + 1, srcv1, dstv1, semi1)

    def pair_body(p, st):
        k0 = start + 2 * p
        more = 2 * p + 2 < cnt
        st = process(k0, st, buf0, buf1, semg0, (semg1, semi1), semi0,
                     True, more)
        st = process(k0 + 1, st, buf1, buf0, semg1, (semg0, semi0), semi1,
                     more, more)
        return st

    init = (zvec,) * 8
    st = lax.fori_loop(jnp.int32(0), npairs, pair_body, init)

    for f in range(4):
        statsv[0, 0, pl.ds(f * 16, 16)] = st[f]
        statsv[0, 0, pl.ds(hh + f * 16, 16)] = st[4 + f]
    wid = c * 16 + s
    pltpu.sync_copy(statsv, stats.at[pl.ds(wid, 1)])

    plsc.subcore_barrier()
    done = 0
    while done < rows_per_tile:
        sz = min(_CHUNK, rows_per_tile - done)
        r0 = base_r + done
        pltpu.sync_copy(acc.at[pl.ds(r0, sz)], dbv0.at[pl.ds(0, sz)])
        pltpu.sync_copy(dbv0.at[pl.ds(0, sz)], ndp.at[c, pl.ds(r0, sz)])
        done += sz


def _sc_edge(src, dst, cep, t1f, ehf):
    n_nodes = ehf.shape[0]
    n_edges = src.shape[0]
    d = ehf.shape[1]
    hh = d // 2
    n_pad = ((n_nodes + 127) // 128) * 128
    mesh = plsc.VectorSubcoreMesh(core_axis_name="c", subcore_axis_name="s")
    assert n_edges % _CHUNK == 0 and (n_edges // _CHUNK) % 2 == 0
    bufset = [
        pltpu.VMEM((_CHUNK,), jnp.int32),
        pltpu.VMEM((_CHUNK,), jnp.int32),
        pltpu.VMEM((_CHUNK,), jnp.int32),
        pltpu.VMEM((_CHUNK, d), F32),
        pltpu.VMEM((_CHUNK, d), F32),
        pltpu.VMEM((_CHUNK, hh), F32),
    ]
    fn = pl.kernel(
        functools.partial(_sc_edge_body, n_nodes, n_pad, n_edges),
        out_type=[
            jax.ShapeDtypeStruct((2, n_edges, hh), F32),
            jax.ShapeDtypeStruct((2, n_pad, d), F32),
            jax.ShapeDtypeStruct((32, 1, d), F32),
        ],
        mesh=mesh,
        scratch_types=bufset + bufset + [
            pltpu.VMEM((1, 1, d), F32),
            pltpu.VMEM_SHARED((n_pad, d), F32),
            pltpu.SemaphoreType.DMA,
            pltpu.SemaphoreType.DMA,
            pltpu.SemaphoreType.DMA,
            pltpu.SemaphoreType.DMA,
        ],
    )
    return fn(src, dst, cep, t1f, ehf)


def kernel(edge_index, nodes_feat, edges_feat, nodes_num_norm_sqrt,
           edges_num_norm_sqrt, emb_h_w, emb_h_b, emb_e_w, emb_e_b,
           Aw, Ab, Bw, Bb, Cw, Cb, Dw, Db, Ew, Eb,
           bn_h_g, bn_h_b, bn_e_g, bn_e_b, readout_w):
    n, d = nodes_feat.shape
    n_edges = edge_index.shape[1]
    num_layers = Aw.shape[0]

    src = edge_index[0].astype(jnp.int32)
    dst = edge_index[1].astype(jnp.int32)
    ces = edges_num_norm_sqrt[0:1, 0:1].astype(F32)
    nnorm = nodes_num_norm_sqrt.astype(F32)

    nblk = 2000
    eblk = 4000

    h = _embed_h(nodes_feat.astype(F32), emb_h_w.astype(F32),
                 emb_h_b.reshape(1, d).astype(F32), nblk)
    e = _embed_e(edges_feat.astype(F32), emb_e_w.astype(F32),
                 emb_e_b.reshape(1, d).astype(F32), eblk)

    for l in range(num_layers):
        ah, t1, ehf = _node_mm(
            h, Aw[l], Ab[l].reshape(1, d), Bw[l], Bb[l].reshape(1, d),
            Dw[l], Db[l].reshape(1, d), Ew[l], Eb[l].reshape(1, d), nblk)
        cep = _ce_mm(e, Cw[l], Cb[l].reshape(1, d), eblk)
        t1f = t1.reshape(2 * n, d)
        enp, ndp, stats = _sc_edge(src, dst, cep, t1f, ehf)
        h, esc, esh = _node_ep(
            ah, ndp, h, nnorm, stats,
            bn_h_g[l].reshape(1, d), bn_h_b[l].reshape(1, d),
            bn_e_g[l].reshape(1, d), bn_e_b[l].reshape(1, d), ces, n_edges)
        e = _edge_ep(e, enp, esc, esh, eblk)

    return _readout(h, readout_w.astype(F32))
